# baseline (speedup 1.0000x reference)
import functools

import jax
import jax.numpy as jnp
from jax import lax
from jax.experimental import pallas as pl
from jax.experimental.pallas import tpu as pltpu
from jax.experimental.pallas import tpu_sc as plsc

N = 10000
NPAD = 10240
E = 320000
D = 128
L = 100000
LPAD = 100352
NW = 32
EPW = E // NW
NPW = NPAD // NW
LPW = LPAD // NW
LANES = 16

f32 = jnp.float32
i32 = jnp.int32


_PREC = jax.lax.Precision.HIGHEST


def _dense_t_body(xd, xp, wldp, wlpd, wlin, t_dp, t_pd):
    w1 = wlin[0:D, 0]
    w2 = wlin[D:2 * D, 0]
    v_dp = jnp.sum(wldp[...] * w2[None, :], axis=1)
    v_pd = jnp.sum(wlpd[...] * w1[None, :], axis=1)
    t_dp[...] = jnp.dot(xd[...], v_dp[:, None], precision=_PREC)[:, 0]
    t_pd[...] = jnp.dot(xp[...], v_pd[:, None], precision=_PREC)[:, 0]


def _dense_self_body(xd, xp, wrdp, wrpd, wlin, bdp, bpd, blin,
                     self_d, self_p):
    w1 = wlin[0:D, 0]
    w2 = wlin[D:2 * D, 0]
    u_dp = jnp.sum(wrdp[...] * w2[None, :], axis=1)
    u_pd = jnp.sum(wrpd[...] * w1[None, :], axis=1)
    c_prot = jnp.sum(bdp[...] * w2)
    c_drug = jnp.sum(bpd[...] * w1) + jnp.sum(blin[...])
    self_d[...] = jnp.dot(xd[...], u_pd[:, None], precision=_PREC)[:, 0] + c_drug
    self_p[...] = jnp.dot(xp[...], u_dp[:, None], precision=_PREC)[:, 0] + c_prot


_dense_t = pl.pallas_call(
    _dense_t_body,
    out_shape=[jax.ShapeDtypeStruct((N,), f32)] * 2,
)

_dense_self = pl.pallas_call(
    _dense_self_body,
    out_shape=[jax.ShapeDtypeStruct((N,), f32)] * 2,
)


_MESH = plsc.VectorSubcoreMesh(core_axis_name="c", subcore_axis_name="s",
                               num_cores=2, num_subcores=16)
_SC_PARAMS = pltpu.CompilerParams(needs_layout_passes=False,
                                  use_tc_tiling_on_sc=False)


def _wid():
    return lax.axis_index("s") * 2 + lax.axis_index("c")


@functools.partial(
    pl.kernel,
    out_type=[jax.ShapeDtypeStruct((NW, NPAD), f32)] * 4,
    mesh=_MESH,
    compiler_params=_SC_PARAMS,
    scratch_types=[
        pltpu.VMEM((EPW,), i32),
        pltpu.VMEM((EPW,), i32),
        pltpu.VMEM((N,), f32),
        pltpu.VMEM((NPAD,), f32),
        pltpu.VMEM((NPAD,), f32),
    ],
)
def _segsum(ei_dp, ei_pd, t_dp, t_pd,
            acc_dp_o, cnt_dp_o, acc_pd_o, cnt_pd_o,
            src_v, dst_v, t_v, acc_v, cnt_v):
    w = _wid()
    zeros = jnp.zeros((LANES,), f32)
    ones = jnp.full((LANES,), 1.0, f32)

    def relation(ei_h, t_h, acc_o, cnt_o):
        pltpu.sync_copy(t_h, t_v)
        pltpu.sync_copy(ei_h.at[0, pl.ds(w * EPW, EPW)], src_v)
        pltpu.sync_copy(ei_h.at[1, pl.ds(w * EPW, EPW)], dst_v)

        def zero_body(i, _):
            acc_v[pl.ds(i * LANES, LANES)] = zeros
            cnt_v[pl.ds(i * LANES, LANES)] = zeros
            return 0
        lax.fori_loop(0, NPAD // LANES, zero_body, 0)

        def body(i, _):
            for u in range(5):
                sl = pl.ds((i * 5 + u) * LANES, LANES)
                d16 = dst_v[sl]
                vals = plsc.load_gather(t_v, [src_v[sl]])
                plsc.addupdate_scatter(acc_v, [d16], vals)
                plsc.addupdate_scatter(cnt_v, [d16], ones)
            return 0
        lax.fori_loop(0, EPW // (LANES * 5), body, 0)
        pltpu.sync_copy(acc_v, acc_o.at[w])
        pltpu.sync_copy(cnt_v, cnt_o.at[w])

    relation(ei_dp, t_dp, acc_dp_o, cnt_dp_o)
    relation(ei_pd, t_pd, acc_pd_o, cnt_pd_o)


@functools.partial(
    pl.kernel,
    out_type=[jax.ShapeDtypeStruct((NPAD,), f32)] * 2,
    mesh=_MESH,
    compiler_params=_SC_PARAMS,
    scratch_types=[
        pltpu.VMEM((NW, NPW), f32),
        pltpu.VMEM((NPW,), f32),
        pltpu.VMEM((NPW,), f32),
        pltpu.VMEM((NPW,), f32),
    ],
)
def _finalize(acc_dp, cnt_dp, acc_pd, cnt_pd, self_p, self_d,
              s_prot_o, s_drug_o,
              part_v, sum_v, self_v, out_v):
    w = _wid()
    nb = w * NPW
    zeros = jnp.zeros((LANES,), f32)

    def side(acc_h, cnt_h, self_h, s_o):
        pltpu.sync_copy(acc_h.at[:, pl.ds(nb, NPW)], part_v)

        def sum_body(c, _):
            v = zeros
            for r in range(NW):
                v = v + part_v[r, pl.ds(c * LANES, LANES)]
            sum_v[pl.ds(c * LANES, LANES)] = v
            return 0
        lax.fori_loop(0, NPW // LANES, sum_body, 0)

        pltpu.sync_copy(cnt_h.at[:, pl.ds(nb, NPW)], part_v)
        pltpu.sync_copy(self_h.at[pl.ds(nb, NPW)], self_v)

        def fin_body(c, _):
            sl = pl.ds(c * LANES, LANES)
            cv = zeros
            for r in range(NW):
                cv = cv + part_v[r, sl]
            out_v[sl] = sum_v[sl] / jnp.maximum(cv, 1.0) + self_v[sl]
            return 0
        lax.fori_loop(0, NPW // LANES, fin_body, 0)
        pltpu.sync_copy(out_v, s_o.at[pl.ds(nb, NPW)])

    side(acc_dp, cnt_dp, self_p, s_prot_o)
    side(acc_pd, cnt_pd, self_d, s_drug_o)


LTAIL = L - 31 * LPW


@functools.partial(
    pl.kernel,
    out_type=jax.ShapeDtypeStruct((L,), f32),
    mesh=_MESH,
    compiler_params=_SC_PARAMS,
    scratch_types=[
        pltpu.VMEM((NPAD,), f32),
        pltpu.VMEM((NPAD,), f32),
        pltpu.VMEM((LPW,), i32),
        pltpu.VMEM((LPW,), i32),
        pltpu.VMEM((LPW,), f32),
    ],
)
def _edge_gather(s_drug, s_prot, eli, out_o,
                 sd_v, sp_v, e0_v, e1_v, o_v):
    w = _wid()
    lb = w * LPW
    pltpu.sync_copy(s_drug, sd_v)
    pltpu.sync_copy(s_prot, sp_v)

    def run(n):
        pltpu.sync_copy(eli.at[0, pl.ds(lb, n)], e0_v.at[pl.ds(0, n)])
        pltpu.sync_copy(eli.at[1, pl.ds(lb, n)], e1_v.at[pl.ds(0, n)])

        def body(i, _):
            sl = pl.ds(i * LANES, LANES)
            o_v[sl] = (plsc.load_gather(sd_v, [e0_v[sl]])
                       + plsc.load_gather(sp_v, [e1_v[sl]]))
            return 0
        lax.fori_loop(0, n // LANES, body, 0)
        pltpu.sync_copy(o_v.at[pl.ds(0, n)], out_o.at[pl.ds(lb, n)])

    @pl.when(w < NW - 1)
    def _():
        run(LPW)

    @pl.when(w == NW - 1)
    def _():
        run(LTAIL)


def kernel(x_drug, x_prot, edge_index_dp, edge_index_pd, edge_label_index,
           Wl_dp, Wr_dp, b_dp, Wl_pd, Wr_pd, b_pd, W_lin, b_lin):
    t_dp, t_pd = _dense_t(x_drug, x_prot, Wl_dp, Wl_pd, W_lin)
    self_d, self_p = _dense_self(
        x_drug, x_prot, Wr_dp, Wr_pd, W_lin, b_dp, b_pd, b_lin)

    self_d_pad = jnp.pad(self_d, (0, NPAD - N))
    self_p_pad = jnp.pad(self_p, (0, NPAD - N))

    acc_dp, cnt_dp, acc_pd, cnt_pd = _segsum(
        edge_index_dp.astype(i32), edge_index_pd.astype(i32), t_dp, t_pd)
    s_prot, s_drug = _finalize(
        acc_dp, cnt_dp, acc_pd, cnt_pd, self_p_pad, self_d_pad)
    out = _edge_gather(s_drug, s_prot, edge_label_index.astype(i32))
    return out[:, None]


# speedup vs baseline: 1.0915x; 1.0915x over previous
import functools

import jax
import jax.numpy as jnp
from jax import lax
from jax.experimental import pallas as pl
from jax.experimental.pallas import tpu as pltpu
from jax.experimental.pallas import tpu_sc as plsc

N = 10000
NPAD = 10240
E = 320000
D = 128
L = 100000
LPAD = 100352
NW = 32
EPW = E // NW
NPW = NPAD // NW
LPW = LPAD // NW
LANES = 16

f32 = jnp.float32
i32 = jnp.int32


_PREC = jax.lax.Precision.HIGHEST


def _dense_t_body(xd, xp, wldp, wlpd, wlin, t_dp, t_pd):
    w1 = wlin[0:D, 0]
    w2 = wlin[D:2 * D, 0]
    v_dp = jnp.sum(wldp[...] * w2[None, :], axis=1)
    v_pd = jnp.sum(wlpd[...] * w1[None, :], axis=1)
    t_dp[...] = jnp.sum(xd[...] * v_dp[None, :], axis=1)
    t_pd[...] = jnp.sum(xp[...] * v_pd[None, :], axis=1)


def _dense_self_body(xd, xp, wrdp, wrpd, wlin, bdp, bpd, blin,
                     self_d, self_p):
    w1 = wlin[0:D, 0]
    w2 = wlin[D:2 * D, 0]
    u_dp = jnp.sum(wrdp[...] * w2[None, :], axis=1)
    u_pd = jnp.sum(wrpd[...] * w1[None, :], axis=1)
    c_prot = jnp.sum(bdp[...] * w2)
    c_drug = jnp.sum(bpd[...] * w1) + jnp.sum(blin[...])
    self_d[...] = jnp.sum(xd[...] * u_pd[None, :], axis=1) + c_drug
    self_p[...] = jnp.sum(xp[...] * u_dp[None, :], axis=1) + c_prot


_dense_t = pl.pallas_call(
    _dense_t_body,
    out_shape=[jax.ShapeDtypeStruct((N,), f32)] * 2,
)

_dense_self = pl.pallas_call(
    _dense_self_body,
    out_shape=[jax.ShapeDtypeStruct((N,), f32)] * 2,
)


_MESH = plsc.VectorSubcoreMesh(core_axis_name="c", subcore_axis_name="s",
                               num_cores=2, num_subcores=16)
_SC_PARAMS = pltpu.CompilerParams(needs_layout_passes=False,
                                  use_tc_tiling_on_sc=False)


def _wid():
    return lax.axis_index("s") * 2 + lax.axis_index("c")


@functools.partial(
    pl.kernel,
    out_type=[jax.ShapeDtypeStruct((NW, NPAD), f32)] * 4,
    mesh=_MESH,
    compiler_params=_SC_PARAMS,
    scratch_types=[
        pltpu.VMEM((EPW,), i32),
        pltpu.VMEM((EPW,), i32),
        pltpu.VMEM((N,), f32),
        pltpu.VMEM((NPAD,), f32),
        pltpu.VMEM((NPAD,), f32),
    ],
)
def _segsum(ei_dp, ei_pd, t_dp, t_pd,
            acc_dp_o, cnt_dp_o, acc_pd_o, cnt_pd_o,
            src_v, dst_v, t_v, acc_v, cnt_v):
    w = _wid()
    zeros = jnp.zeros((LANES,), f32)
    ones = jnp.full((LANES,), 1.0, f32)

    def relation(ei_h, t_h, acc_o, cnt_o):
        pltpu.sync_copy(t_h, t_v)
        pltpu.sync_copy(ei_h.at[0, pl.ds(w * EPW, EPW)], src_v)
        pltpu.sync_copy(ei_h.at[1, pl.ds(w * EPW, EPW)], dst_v)

        def zero_body(i, _):
            acc_v[pl.ds(i * LANES, LANES)] = zeros
            cnt_v[pl.ds(i * LANES, LANES)] = zeros
            return 0
        lax.fori_loop(0, NPAD // LANES, zero_body, 0)

        def body(i, _):
            for u in range(5):
                sl = pl.ds((i * 5 + u) * LANES, LANES)
                d16 = dst_v[sl]
                vals = plsc.load_gather(t_v, [src_v[sl]])
                plsc.addupdate_scatter(acc_v, [d16], vals)
                plsc.addupdate_scatter(cnt_v, [d16], ones)
            return 0
        lax.fori_loop(0, EPW // (LANES * 5), body, 0)
        pltpu.sync_copy(acc_v, acc_o.at[w])
        pltpu.sync_copy(cnt_v, cnt_o.at[w])

    relation(ei_dp, t_dp, acc_dp_o, cnt_dp_o)
    relation(ei_pd, t_pd, acc_pd_o, cnt_pd_o)


@functools.partial(
    pl.kernel,
    out_type=[jax.ShapeDtypeStruct((NPAD,), f32)] * 2,
    mesh=_MESH,
    compiler_params=_SC_PARAMS,
    scratch_types=[
        pltpu.VMEM((NW, NPW), f32),
        pltpu.VMEM((NPW,), f32),
        pltpu.VMEM((NPW,), f32),
        pltpu.VMEM((NPW,), f32),
    ],
)
def _finalize(acc_dp, cnt_dp, acc_pd, cnt_pd, self_p, self_d,
              s_prot_o, s_drug_o,
              part_v, sum_v, self_v, out_v):
    w = _wid()
    nb = w * NPW
    zeros = jnp.zeros((LANES,), f32)

    def side(acc_h, cnt_h, self_h, s_o):
        pltpu.sync_copy(acc_h.at[:, pl.ds(nb, NPW)], part_v)

        def sum_body(c, _):
            v = zeros
            for r in range(NW):
                v = v + part_v[r, pl.ds(c * LANES, LANES)]
            sum_v[pl.ds(c * LANES, LANES)] = v
            return 0
        lax.fori_loop(0, NPW // LANES, sum_body, 0)

        pltpu.sync_copy(cnt_h.at[:, pl.ds(nb, NPW)], part_v)
        pltpu.sync_copy(self_h.at[pl.ds(nb, NPW)], self_v)

        def fin_body(c, _):
            sl = pl.ds(c * LANES, LANES)
            cv = zeros
            for r in range(NW):
                cv = cv + part_v[r, sl]
            out_v[sl] = sum_v[sl] / jnp.maximum(cv, 1.0) + self_v[sl]
            return 0
        lax.fori_loop(0, NPW // LANES, fin_body, 0)
        pltpu.sync_copy(out_v, s_o.at[pl.ds(nb, NPW)])

    side(acc_dp, cnt_dp, self_p, s_prot_o)
    side(acc_pd, cnt_pd, self_d, s_drug_o)


LTAIL = L - 31 * LPW


@functools.partial(
    pl.kernel,
    out_type=jax.ShapeDtypeStruct((L,), f32),
    mesh=_MESH,
    compiler_params=_SC_PARAMS,
    scratch_types=[
        pltpu.VMEM((NPAD,), f32),
        pltpu.VMEM((NPAD,), f32),
        pltpu.VMEM((LPW,), i32),
        pltpu.VMEM((LPW,), i32),
        pltpu.VMEM((LPW,), f32),
    ],
)
def _edge_gather(s_drug, s_prot, eli, out_o,
                 sd_v, sp_v, e0_v, e1_v, o_v):
    w = _wid()
    lb = w * LPW
    pltpu.sync_copy(s_drug, sd_v)
    pltpu.sync_copy(s_prot, sp_v)

    def run(n):
        pltpu.sync_copy(eli.at[0, pl.ds(lb, n)], e0_v.at[pl.ds(0, n)])
        pltpu.sync_copy(eli.at[1, pl.ds(lb, n)], e1_v.at[pl.ds(0, n)])

        def body(i, _):
            sl = pl.ds(i * LANES, LANES)
            o_v[sl] = (plsc.load_gather(sd_v, [e0_v[sl]])
                       + plsc.load_gather(sp_v, [e1_v[sl]]))
            return 0
        lax.fori_loop(0, n // LANES, body, 0)
        pltpu.sync_copy(o_v.at[pl.ds(0, n)], out_o.at[pl.ds(lb, n)])

    @pl.when(w < NW - 1)
    def _():
        run(LPW)

    @pl.when(w == NW - 1)
    def _():
        run(LTAIL)


def kernel(x_drug, x_prot, edge_index_dp, edge_index_pd, edge_label_index,
           Wl_dp, Wr_dp, b_dp, Wl_pd, Wr_pd, b_pd, W_lin, b_lin):
    t_dp, t_pd = _dense_t(x_drug, x_prot, Wl_dp, Wl_pd, W_lin)
    self_d, self_p = _dense_self(
        x_drug, x_prot, Wr_dp, Wr_pd, W_lin, b_dp, b_pd, b_lin)

    self_d_pad = jnp.pad(self_d, (0, NPAD - N))
    self_p_pad = jnp.pad(self_p, (0, NPAD - N))

    acc_dp, cnt_dp, acc_pd, cnt_pd = _segsum(
        edge_index_dp.astype(i32), edge_index_pd.astype(i32), t_dp, t_pd)
    s_prot, s_drug = _finalize(
        acc_dp, cnt_dp, acc_pd, cnt_pd, self_p_pad, self_d_pad)
    out = _edge_gather(s_drug, s_prot, edge_label_index.astype(i32))
    return out[:, None]


# speedup vs baseline: 1.2662x; 1.1601x over previous
import functools

import jax
import jax.numpy as jnp
from jax import lax
from jax.experimental import pallas as pl
from jax.experimental.pallas import tpu as pltpu
from jax.experimental.pallas import tpu_sc as plsc

N = 10000
NPAD = 10240
E = 320000
D = 128
L = 100000
LPAD = 100352
NW = 32
EPW = E // NW
NPW = NPAD // NW
LPW = LPAD // NW
LANES = 16

f32 = jnp.float32
i32 = jnp.int32


_PREC = jax.lax.Precision.HIGHEST


def _dense_t_body(xd, xp, wldp, wlpd, wlin, t_dp, t_pd):
    w1 = wlin[0:D, 0]
    w2 = wlin[D:2 * D, 0]
    v_dp = jnp.sum(wldp[...] * w2[None, :], axis=1)
    v_pd = jnp.sum(wlpd[...] * w1[None, :], axis=1)
    t_dp[...] = jnp.sum(xd[...] * v_dp[None, :], axis=1)
    t_pd[...] = jnp.sum(xp[...] * v_pd[None, :], axis=1)


def _dense_self_body(xd, xp, wrdp, wrpd, wlin, bdp, bpd, blin,
                     self_d, self_p):
    w1 = wlin[0:D, 0]
    w2 = wlin[D:2 * D, 0]
    u_dp = jnp.sum(wrdp[...] * w2[None, :], axis=1)
    u_pd = jnp.sum(wrpd[...] * w1[None, :], axis=1)
    c_prot = jnp.sum(bdp[...] * w2)
    c_drug = jnp.sum(bpd[...] * w1) + jnp.sum(blin[...])
    self_d[...] = jnp.sum(xd[...] * u_pd[None, :], axis=1) + c_drug
    self_p[...] = jnp.sum(xp[...] * u_dp[None, :], axis=1) + c_prot


_dense_t = pl.pallas_call(
    _dense_t_body,
    out_shape=[jax.ShapeDtypeStruct((N,), f32)] * 2,
)

_dense_self = pl.pallas_call(
    _dense_self_body,
    out_shape=[jax.ShapeDtypeStruct((N,), f32)] * 2,
)


_MESH = plsc.VectorSubcoreMesh(core_axis_name="c", subcore_axis_name="s",
                               num_cores=2, num_subcores=16)
_SC_PARAMS = pltpu.CompilerParams(needs_layout_passes=False,
                                  use_tc_tiling_on_sc=False)


def _wid():
    return lax.axis_index("s") * 2 + lax.axis_index("c")


@functools.partial(
    pl.kernel,
    out_type=[jax.ShapeDtypeStruct((NW, NPAD), f32)] * 4,
    mesh=_MESH,
    compiler_params=_SC_PARAMS,
    scratch_types=[
        pltpu.VMEM((EPW,), i32),
        pltpu.VMEM((EPW,), i32),
        pltpu.VMEM((EPW,), i32),
        pltpu.VMEM((EPW,), i32),
        pltpu.VMEM((N,), f32),
        pltpu.VMEM((N,), f32),
        pltpu.VMEM((NPAD,), f32),
        pltpu.VMEM((NPAD,), f32),
        pltpu.VMEM((NPAD,), f32),
        pltpu.VMEM((NPAD,), f32),
        pltpu.SemaphoreType.DMA,
    ],
)
def _segsum(ei_dp, ei_pd, t_dp, t_pd,
            acc_dp_o, cnt_dp_o, acc_pd_o, cnt_pd_o,
            src1_v, dst1_v, src2_v, dst2_v, t1_v, t2_v,
            acc1_v, cnt1_v, acc2_v, cnt2_v, sem):
    w = _wid()
    eb = pl.ds(w * EPW, EPW)
    zeros = jnp.zeros((LANES,), f32)
    ones = jnp.full((LANES,), 1.0, f32)

    copies = [
        pltpu.async_copy(t_dp, t1_v, sem),
        pltpu.async_copy(t_pd, t2_v, sem),
        pltpu.async_copy(ei_dp.at[0, eb], src1_v, sem),
        pltpu.async_copy(ei_dp.at[1, eb], dst1_v, sem),
        pltpu.async_copy(ei_pd.at[0, eb], src2_v, sem),
        pltpu.async_copy(ei_pd.at[1, eb], dst2_v, sem),
    ]

    def zero_body(i, _):
        sl = pl.ds(i * LANES, LANES)
        acc1_v[sl] = zeros
        cnt1_v[sl] = zeros
        acc2_v[sl] = zeros
        cnt2_v[sl] = zeros
        return 0
    lax.fori_loop(0, NPAD // LANES, zero_body, 0)
    for c in copies:
        c.wait()

    def relation(src_v, dst_v, t_v, acc_v, cnt_v):
        @plsc.parallel_loop(0, EPW // LANES, unroll=5)
        def _(i):
            sl = pl.ds(i * LANES, LANES)
            d16 = dst_v[sl]
            vals = plsc.load_gather(t_v, [src_v[sl]])
            plsc.addupdate_scatter(acc_v, [d16], vals)
            plsc.addupdate_scatter(cnt_v, [d16], ones)

    relation(src1_v, dst1_v, t1_v, acc1_v, cnt1_v)
    relation(src2_v, dst2_v, t2_v, acc2_v, cnt2_v)

    outs = [
        pltpu.async_copy(acc1_v, acc_dp_o.at[w], sem),
        pltpu.async_copy(cnt1_v, cnt_dp_o.at[w], sem),
        pltpu.async_copy(acc2_v, acc_pd_o.at[w], sem),
        pltpu.async_copy(cnt2_v, cnt_pd_o.at[w], sem),
    ]
    for c in outs:
        c.wait()


@functools.partial(
    pl.kernel,
    out_type=[jax.ShapeDtypeStruct((NPAD,), f32)] * 2,
    mesh=_MESH,
    compiler_params=_SC_PARAMS,
    scratch_types=[
        pltpu.VMEM((NW, NPW), f32),
        pltpu.VMEM((NPW,), f32),
        pltpu.VMEM((NPW,), f32),
        pltpu.VMEM((NPW,), f32),
    ],
)
def _finalize(acc_dp, cnt_dp, acc_pd, cnt_pd, self_p, self_d,
              s_prot_o, s_drug_o,
              part_v, sum_v, self_v, out_v):
    w = _wid()
    nb = w * NPW
    zeros = jnp.zeros((LANES,), f32)

    def side(acc_h, cnt_h, self_h, s_o):
        pltpu.sync_copy(acc_h.at[:, pl.ds(nb, NPW)], part_v)

        def sum_body(c, _):
            v = zeros
            for r in range(NW):
                v = v + part_v[r, pl.ds(c * LANES, LANES)]
            sum_v[pl.ds(c * LANES, LANES)] = v
            return 0
        lax.fori_loop(0, NPW // LANES, sum_body, 0)

        pltpu.sync_copy(cnt_h.at[:, pl.ds(nb, NPW)], part_v)
        pltpu.sync_copy(self_h.at[pl.ds(nb, NPW)], self_v)

        def fin_body(c, _):
            sl = pl.ds(c * LANES, LANES)
            cv = zeros
            for r in range(NW):
                cv = cv + part_v[r, sl]
            out_v[sl] = sum_v[sl] / jnp.maximum(cv, 1.0) + self_v[sl]
            return 0
        lax.fori_loop(0, NPW // LANES, fin_body, 0)
        pltpu.sync_copy(out_v, s_o.at[pl.ds(nb, NPW)])

    side(acc_dp, cnt_dp, self_p, s_prot_o)
    side(acc_pd, cnt_pd, self_d, s_drug_o)


LTAIL = L - 31 * LPW


@functools.partial(
    pl.kernel,
    out_type=jax.ShapeDtypeStruct((L,), f32),
    mesh=_MESH,
    compiler_params=_SC_PARAMS,
    scratch_types=[
        pltpu.VMEM((NPAD,), f32),
        pltpu.VMEM((NPAD,), f32),
        pltpu.VMEM((LPW,), i32),
        pltpu.VMEM((LPW,), i32),
        pltpu.VMEM((LPW,), f32),
    ],
)
def _edge_gather(s_drug, s_prot, eli, out_o,
                 sd_v, sp_v, e0_v, e1_v, o_v):
    w = _wid()
    lb = w * LPW
    pltpu.sync_copy(s_drug, sd_v)
    pltpu.sync_copy(s_prot, sp_v)

    def run(n):
        pltpu.sync_copy(eli.at[0, pl.ds(lb, n)], e0_v.at[pl.ds(0, n)])
        pltpu.sync_copy(eli.at[1, pl.ds(lb, n)], e1_v.at[pl.ds(0, n)])

        def body(i, _):
            sl = pl.ds(i * LANES, LANES)
            o_v[sl] = (plsc.load_gather(sd_v, [e0_v[sl]])
                       + plsc.load_gather(sp_v, [e1_v[sl]]))
            return 0
        lax.fori_loop(0, n // LANES, body, 0)
        pltpu.sync_copy(o_v.at[pl.ds(0, n)], out_o.at[pl.ds(lb, n)])

    @pl.when(w < NW - 1)
    def _():
        run(LPW)

    @pl.when(w == NW - 1)
    def _():
        run(LTAIL)


def kernel(x_drug, x_prot, edge_index_dp, edge_index_pd, edge_label_index,
           Wl_dp, Wr_dp, b_dp, Wl_pd, Wr_pd, b_pd, W_lin, b_lin):
    t_dp, t_pd = _dense_t(x_drug, x_prot, Wl_dp, Wl_pd, W_lin)
    self_d, self_p = _dense_self(
        x_drug, x_prot, Wr_dp, Wr_pd, W_lin, b_dp, b_pd, b_lin)

    self_d_pad = jnp.pad(self_d, (0, NPAD - N))
    self_p_pad = jnp.pad(self_p, (0, NPAD - N))

    acc_dp, cnt_dp, acc_pd, cnt_pd = _segsum(
        edge_index_dp.astype(i32), edge_index_pd.astype(i32), t_dp, t_pd)
    s_prot, s_drug = _finalize(
        acc_dp, cnt_dp, acc_pd, cnt_pd, self_p_pad, self_d_pad)
    out = _edge_gather(s_drug, s_prot, edge_label_index.astype(i32))
    return out[:, None]


# speedup vs baseline: 1.3334x; 1.0531x over previous
import functools

import jax
import jax.numpy as jnp
from jax import lax
from jax.experimental import pallas as pl
from jax.experimental.pallas import tpu as pltpu
from jax.experimental.pallas import tpu_sc as plsc

N = 10000
NPAD = 10240
E = 320000
D = 128
L = 100000
LPAD = 100352
NW = 32
EPW = E // NW
NPW = NPAD // NW
LPW = LPAD // NW
LANES = 16

f32 = jnp.float32
i32 = jnp.int32


_PREC = jax.lax.Precision.HIGHEST


def _dense_t_body(xd, xp, wldp, wlpd, wlin, t_dp, t_pd):
    w1 = wlin[0:D, 0]
    w2 = wlin[D:2 * D, 0]
    v_dp = jnp.sum(wldp[...] * w2[None, :], axis=1)
    v_pd = jnp.sum(wlpd[...] * w1[None, :], axis=1)
    t_dp[...] = jnp.sum(xd[...] * v_dp[None, :], axis=1)
    t_pd[...] = jnp.sum(xp[...] * v_pd[None, :], axis=1)


def _dense_self_body(xd, xp, wrdp, wrpd, wlin, bdp, bpd, blin,
                     self_d, self_p):
    w1 = wlin[0:D, 0]
    w2 = wlin[D:2 * D, 0]
    u_dp = jnp.sum(wrdp[...] * w2[None, :], axis=1)
    u_pd = jnp.sum(wrpd[...] * w1[None, :], axis=1)
    c_prot = jnp.sum(bdp[...] * w2)
    c_drug = jnp.sum(bpd[...] * w1) + jnp.sum(blin[...])
    self_d[...] = jnp.sum(xd[...] * u_pd[None, :], axis=1) + c_drug
    self_p[...] = jnp.sum(xp[...] * u_dp[None, :], axis=1) + c_prot


_dense_t = pl.pallas_call(
    _dense_t_body,
    out_shape=[jax.ShapeDtypeStruct((N,), f32)] * 2,
)

_dense_self = pl.pallas_call(
    _dense_self_body,
    out_shape=[jax.ShapeDtypeStruct((N,), f32)] * 2,
)


_MESH = plsc.VectorSubcoreMesh(core_axis_name="c", subcore_axis_name="s",
                               num_cores=2, num_subcores=16)
_SC_PARAMS = pltpu.CompilerParams(needs_layout_passes=False,
                                  use_tc_tiling_on_sc=False)


def _wid():
    return lax.axis_index("s") * 2 + lax.axis_index("c")


@functools.partial(
    pl.kernel,
    out_type=[jax.ShapeDtypeStruct((NW, NPAD), f32)] * 4,
    mesh=_MESH,
    compiler_params=_SC_PARAMS,
    scratch_types=[
        pltpu.VMEM((EPW,), i32),
        pltpu.VMEM((EPW,), i32),
        pltpu.VMEM((EPW,), i32),
        pltpu.VMEM((EPW,), i32),
        pltpu.VMEM((N,), f32),
        pltpu.VMEM((N,), f32),
        pltpu.VMEM((NPAD,), f32),
        pltpu.VMEM((NPAD,), f32),
        pltpu.VMEM((NPAD,), f32),
        pltpu.VMEM((NPAD,), f32),
        pltpu.SemaphoreType.DMA,
    ],
)
def _segsum(ei_dp, ei_pd, t_dp, t_pd,
            acc_dp_o, cnt_dp_o, acc_pd_o, cnt_pd_o,
            src1_v, dst1_v, src2_v, dst2_v, t1_v, t2_v,
            acc1_v, cnt1_v, acc2_v, cnt2_v, sem):
    w = _wid()
    eb = pl.ds(w * EPW, EPW)
    zeros = jnp.zeros((LANES,), f32)
    ones = jnp.full((LANES,), 1.0, f32)

    copies = [
        pltpu.async_copy(t_dp, t1_v, sem),
        pltpu.async_copy(t_pd, t2_v, sem),
        pltpu.async_copy(ei_dp.at[0, eb], src1_v, sem),
        pltpu.async_copy(ei_dp.at[1, eb], dst1_v, sem),
        pltpu.async_copy(ei_pd.at[0, eb], src2_v, sem),
        pltpu.async_copy(ei_pd.at[1, eb], dst2_v, sem),
    ]

    def zero_body(i, _):
        sl = pl.ds(i * LANES, LANES)
        acc1_v[sl] = zeros
        cnt1_v[sl] = zeros
        acc2_v[sl] = zeros
        cnt2_v[sl] = zeros
        return 0
    lax.fori_loop(0, NPAD // LANES, zero_body, 0)
    for c in copies:
        c.wait()

    def relation(src_v, dst_v, t_v, acc_v, cnt_v):
        @plsc.parallel_loop(0, EPW // LANES, unroll=5)
        def _(i):
            sl = pl.ds(i * LANES, LANES)
            d16 = dst_v[sl]
            vals = plsc.load_gather(t_v, [src_v[sl]])
            plsc.addupdate_scatter(acc_v, [d16], vals)
            plsc.addupdate_scatter(cnt_v, [d16], ones)

    relation(src1_v, dst1_v, t1_v, acc1_v, cnt1_v)
    relation(src2_v, dst2_v, t2_v, acc2_v, cnt2_v)

    outs = [
        pltpu.async_copy(acc1_v, acc_dp_o.at[w], sem),
        pltpu.async_copy(cnt1_v, cnt_dp_o.at[w], sem),
        pltpu.async_copy(acc2_v, acc_pd_o.at[w], sem),
        pltpu.async_copy(cnt2_v, cnt_pd_o.at[w], sem),
    ]
    for c in outs:
        c.wait()


@functools.partial(
    pl.kernel,
    out_type=[jax.ShapeDtypeStruct((NPAD,), f32)] * 2,
    mesh=_MESH,
    compiler_params=_SC_PARAMS,
    scratch_types=[
        pltpu.VMEM((NW, NPW), f32),
        pltpu.VMEM((NW, NPW), f32),
        pltpu.VMEM((NW, NPW), f32),
        pltpu.VMEM((NW, NPW), f32),
        pltpu.VMEM((NPW,), f32),
        pltpu.VMEM((NPW,), f32),
        pltpu.VMEM((NPW,), f32),
        pltpu.VMEM((NPW,), f32),
        pltpu.SemaphoreType.DMA,
    ],
)
def _finalize(acc_dp, cnt_dp, acc_pd, cnt_pd, self_p, self_d,
              s_prot_o, s_drug_o,
              acc1_v, cnt1_v, acc2_v, cnt2_v, self1_v, self2_v,
              out1_v, out2_v, sem):
    w = _wid()
    nb = w * NPW
    nsl = pl.ds(nb, NPW)
    zeros = jnp.zeros((LANES,), f32)

    copies = [
        pltpu.async_copy(acc_dp.at[:, nsl], acc1_v, sem),
        pltpu.async_copy(cnt_dp.at[:, nsl], cnt1_v, sem),
        pltpu.async_copy(acc_pd.at[:, nsl], acc2_v, sem),
        pltpu.async_copy(cnt_pd.at[:, nsl], cnt2_v, sem),
        pltpu.async_copy(self_p.at[nsl], self1_v, sem),
        pltpu.async_copy(self_d.at[nsl], self2_v, sem),
    ]
    for c in copies:
        c.wait()

    def side(acc_v, cnt_v, self_v, out_v):
        @plsc.parallel_loop(0, NPW // LANES, unroll=4)
        def _(c):
            sl = pl.ds(c * LANES, LANES)
            sv = zeros
            cv = zeros
            for r in range(NW):
                sv = sv + acc_v[r, sl]
                cv = cv + cnt_v[r, sl]
            out_v[sl] = sv / jnp.maximum(cv, 1.0) + self_v[sl]

    side(acc1_v, cnt1_v, self1_v, out1_v)
    side(acc2_v, cnt2_v, self2_v, out2_v)
    o1 = pltpu.async_copy(out1_v, s_prot_o.at[nsl], sem)
    o2 = pltpu.async_copy(out2_v, s_drug_o.at[nsl], sem)
    o1.wait()
    o2.wait()


LTAIL = L - 31 * LPW


@functools.partial(
    pl.kernel,
    out_type=jax.ShapeDtypeStruct((L,), f32),
    mesh=_MESH,
    compiler_params=_SC_PARAMS,
    scratch_types=[
        pltpu.VMEM((NPAD,), f32),
        pltpu.VMEM((NPAD,), f32),
        pltpu.VMEM((LPW,), i32),
        pltpu.VMEM((LPW,), i32),
        pltpu.VMEM((LPW,), f32),
        pltpu.SemaphoreType.DMA,
    ],
)
def _edge_gather(s_drug, s_prot, eli, out_o,
                 sd_v, sp_v, e0_v, e1_v, o_v, sem):
    w = _wid()
    lb = w * LPW

    def run(n):
        copies = [
            pltpu.async_copy(s_drug, sd_v, sem),
            pltpu.async_copy(s_prot, sp_v, sem),
            pltpu.async_copy(eli.at[0, pl.ds(lb, n)], e0_v.at[pl.ds(0, n)], sem),
            pltpu.async_copy(eli.at[1, pl.ds(lb, n)], e1_v.at[pl.ds(0, n)], sem),
        ]
        for c in copies:
            c.wait()

        @plsc.parallel_loop(0, n // LANES, unroll=2)
        def _(i):
            sl = pl.ds(i * LANES, LANES)
            o_v[sl] = (plsc.load_gather(sd_v, [e0_v[sl]])
                       + plsc.load_gather(sp_v, [e1_v[sl]]))
        pltpu.sync_copy(o_v.at[pl.ds(0, n)], out_o.at[pl.ds(lb, n)])

    @pl.when(w < NW - 1)
    def _():
        run(LPW)

    @pl.when(w == NW - 1)
    def _():
        run(LTAIL)


def kernel(x_drug, x_prot, edge_index_dp, edge_index_pd, edge_label_index,
           Wl_dp, Wr_dp, b_dp, Wl_pd, Wr_pd, b_pd, W_lin, b_lin):
    t_dp, t_pd = _dense_t(x_drug, x_prot, Wl_dp, Wl_pd, W_lin)
    self_d, self_p = _dense_self(
        x_drug, x_prot, Wr_dp, Wr_pd, W_lin, b_dp, b_pd, b_lin)

    self_d_pad = jnp.pad(self_d, (0, NPAD - N))
    self_p_pad = jnp.pad(self_p, (0, NPAD - N))

    acc_dp, cnt_dp, acc_pd, cnt_pd = _segsum(
        edge_index_dp.astype(i32), edge_index_pd.astype(i32), t_dp, t_pd)
    s_prot, s_drug = _finalize(
        acc_dp, cnt_dp, acc_pd, cnt_pd, self_p_pad, self_d_pad)
    out = _edge_gather(s_drug, s_prot, edge_label_index.astype(i32))
    return out[:, None]


# speedup vs baseline: 1.3822x; 1.0366x over previous
import functools

import jax
import jax.numpy as jnp
from jax import lax
from jax.experimental import pallas as pl
from jax.experimental.pallas import tpu as pltpu
from jax.experimental.pallas import tpu_sc as plsc

N = 10000
NPAD = 10240
E = 320000
D = 128
L = 100000
LPAD = 100352
NW = 32
EPW = E // NW
NPW = NPAD // NW
LPW = LPAD // NW
LANES = 16

f32 = jnp.float32
i32 = jnp.int32


_PREC = jax.lax.Precision.HIGHEST
_DN = (((1,), (1,)), ((), ()))


def _dense8_body(xd, xp, wldp, wlpd, wrdp, wrpd, wlin, bdp, bpd, blin, t8):
    w1 = wlin[0:D, 0]
    w2 = wlin[D:2 * D, 0]
    v_dp = jnp.sum(wldp[...] * w2[None, :], axis=1)
    v_pd = jnp.sum(wlpd[...] * w1[None, :], axis=1)
    u_pd = jnp.sum(wrpd[...] * w1[None, :], axis=1)
    u_dp = jnp.sum(wrdp[...] * w2[None, :], axis=1)
    z1 = jnp.zeros((1, D), f32)
    z4 = jnp.zeros((4, D), f32)
    a = jnp.concatenate([v_dp[None, :], z1, u_pd[None, :], z1, z4], axis=0)
    b = jnp.concatenate([z1, v_pd[None, :], z1, u_dp[None, :], z4], axis=0)
    r = (lax.dot_general(a, xd[...], _DN, precision=_PREC)
         + lax.dot_general(b, xp[...], _DN, precision=_PREC))
    c_drug = jnp.sum(bpd[...] * w1) + jnp.sum(blin[...])
    c_prot = jnp.sum(bdp[...] * w2)
    rowid = lax.broadcasted_iota(i32, (8, 1), 0)
    bias = (jnp.where(rowid == 2, c_drug, 0.0)
            + jnp.where(rowid == 3, c_prot, 0.0))
    t8[...] = r + bias


_dense8 = pl.pallas_call(
    _dense8_body,
    out_shape=jax.ShapeDtypeStruct((8, N), f32),
)


_MESH = plsc.VectorSubcoreMesh(core_axis_name="c", subcore_axis_name="s",
                               num_cores=2, num_subcores=16)
_SC_PARAMS = pltpu.CompilerParams(needs_layout_passes=False,
                                  use_tc_tiling_on_sc=False)


def _wid():
    return lax.axis_index("s") * 2 + lax.axis_index("c")


@functools.partial(
    pl.kernel,
    out_type=[jax.ShapeDtypeStruct((NW, NPAD), f32)] * 4,
    mesh=_MESH,
    compiler_params=_SC_PARAMS,
    scratch_types=[
        pltpu.VMEM((EPW,), i32),
        pltpu.VMEM((EPW,), i32),
        pltpu.VMEM((EPW,), i32),
        pltpu.VMEM((EPW,), i32),
        pltpu.VMEM((N,), f32),
        pltpu.VMEM((N,), f32),
        pltpu.VMEM((NPAD,), f32),
        pltpu.VMEM((NPAD,), f32),
        pltpu.VMEM((NPAD,), f32),
        pltpu.VMEM((NPAD,), f32),
        pltpu.SemaphoreType.DMA,
    ],
)
def _segsum(ei_dp, ei_pd, t8,
            acc_dp_o, cnt_dp_o, acc_pd_o, cnt_pd_o,
            src1_v, dst1_v, src2_v, dst2_v, t1_v, t2_v,
            acc1_v, cnt1_v, acc2_v, cnt2_v, sem):
    w = _wid()
    eb = pl.ds(w * EPW, EPW)
    zeros = jnp.zeros((LANES,), f32)
    ones = jnp.full((LANES,), 1.0, f32)

    copies = [
        pltpu.async_copy(t8.at[0], t1_v, sem),
        pltpu.async_copy(t8.at[1], t2_v, sem),
        pltpu.async_copy(ei_dp.at[0, eb], src1_v, sem),
        pltpu.async_copy(ei_dp.at[1, eb], dst1_v, sem),
        pltpu.async_copy(ei_pd.at[0, eb], src2_v, sem),
        pltpu.async_copy(ei_pd.at[1, eb], dst2_v, sem),
    ]

    def zero_body(i, _):
        sl = pl.ds(i * LANES, LANES)
        acc1_v[sl] = zeros
        cnt1_v[sl] = zeros
        acc2_v[sl] = zeros
        cnt2_v[sl] = zeros
        return 0
    lax.fori_loop(0, NPAD // LANES, zero_body, 0)
    for c in copies:
        c.wait()

    def relation(src_v, dst_v, t_v, acc_v, cnt_v):
        @plsc.parallel_loop(0, EPW // LANES, unroll=5)
        def _(i):
            sl = pl.ds(i * LANES, LANES)
            d16 = dst_v[sl]
            vals = plsc.load_gather(t_v, [src_v[sl]])
            plsc.addupdate_scatter(acc_v, [d16], vals)
            plsc.addupdate_scatter(cnt_v, [d16], ones)

    relation(src1_v, dst1_v, t1_v, acc1_v, cnt1_v)
    relation(src2_v, dst2_v, t2_v, acc2_v, cnt2_v)

    outs = [
        pltpu.async_copy(acc1_v, acc_dp_o.at[w], sem),
        pltpu.async_copy(cnt1_v, cnt_dp_o.at[w], sem),
        pltpu.async_copy(acc2_v, acc_pd_o.at[w], sem),
        pltpu.async_copy(cnt2_v, cnt_pd_o.at[w], sem),
    ]
    for c in outs:
        c.wait()


@functools.partial(
    pl.kernel,
    out_type=[jax.ShapeDtypeStruct((NPAD,), f32)] * 2,
    mesh=_MESH,
    compiler_params=_SC_PARAMS,
    scratch_types=[
        pltpu.VMEM((NW, NPW), f32),
        pltpu.VMEM((NW, NPW), f32),
        pltpu.VMEM((NW, NPW), f32),
        pltpu.VMEM((NW, NPW), f32),
        pltpu.VMEM((NPW,), f32),
        pltpu.VMEM((NPW,), f32),
        pltpu.VMEM((NPW,), f32),
        pltpu.VMEM((NPW,), f32),
        pltpu.SemaphoreType.DMA,
    ],
)
def _finalize(acc_dp, cnt_dp, acc_pd, cnt_pd, t8,
              s_prot_o, s_drug_o,
              acc1_v, cnt1_v, acc2_v, cnt2_v, self1_v, self2_v,
              out1_v, out2_v, sem):
    w = _wid()
    nb = w * NPW
    nsl = pl.ds(nb, NPW)
    zeros = jnp.zeros((LANES,), f32)

    copies = [
        pltpu.async_copy(acc_dp.at[:, nsl], acc1_v, sem),
        pltpu.async_copy(cnt_dp.at[:, nsl], cnt1_v, sem),
        pltpu.async_copy(acc_pd.at[:, nsl], acc2_v, sem),
        pltpu.async_copy(cnt_pd.at[:, nsl], cnt2_v, sem),
        pltpu.async_copy(t8.at[3, nsl], self1_v, sem),
        pltpu.async_copy(t8.at[2, nsl], self2_v, sem),
    ]
    for c in copies:
        c.wait()

    def side(acc_v, cnt_v, self_v, out_v):
        @plsc.parallel_loop(0, NPW // LANES, unroll=4)
        def _(c):
            sl = pl.ds(c * LANES, LANES)
            sv = zeros
            cv = zeros
            for r in range(NW):
                sv = sv + acc_v[r, sl]
                cv = cv + cnt_v[r, sl]
            out_v[sl] = sv / jnp.maximum(cv, 1.0) + self_v[sl]

    side(acc1_v, cnt1_v, self1_v, out1_v)
    side(acc2_v, cnt2_v, self2_v, out2_v)
    o1 = pltpu.async_copy(out1_v, s_prot_o.at[nsl], sem)
    o2 = pltpu.async_copy(out2_v, s_drug_o.at[nsl], sem)
    o1.wait()
    o2.wait()


LTAIL = L - 31 * LPW


@functools.partial(
    pl.kernel,
    out_type=jax.ShapeDtypeStruct((L,), f32),
    mesh=_MESH,
    compiler_params=_SC_PARAMS,
    scratch_types=[
        pltpu.VMEM((NPAD,), f32),
        pltpu.VMEM((NPAD,), f32),
        pltpu.VMEM((LPW,), i32),
        pltpu.VMEM((LPW,), i32),
        pltpu.VMEM((LPW,), f32),
        pltpu.SemaphoreType.DMA,
    ],
)
def _edge_gather(s_drug, s_prot, eli, out_o,
                 sd_v, sp_v, e0_v, e1_v, o_v, sem):
    w = _wid()
    lb = w * LPW

    def run(n):
        copies = [
            pltpu.async_copy(s_drug, sd_v, sem),
            pltpu.async_copy(s_prot, sp_v, sem),
            pltpu.async_copy(eli.at[0, pl.ds(lb, n)], e0_v.at[pl.ds(0, n)], sem),
            pltpu.async_copy(eli.at[1, pl.ds(lb, n)], e1_v.at[pl.ds(0, n)], sem),
        ]
        for c in copies:
            c.wait()

        @plsc.parallel_loop(0, n // LANES, unroll=2)
        def _(i):
            sl = pl.ds(i * LANES, LANES)
            o_v[sl] = (plsc.load_gather(sd_v, [e0_v[sl]])
                       + plsc.load_gather(sp_v, [e1_v[sl]]))
        pltpu.sync_copy(o_v.at[pl.ds(0, n)], out_o.at[pl.ds(lb, n)])

    @pl.when(w < NW - 1)
    def _():
        run(LPW)

    @pl.when(w == NW - 1)
    def _():
        run(LTAIL)


def kernel(x_drug, x_prot, edge_index_dp, edge_index_pd, edge_label_index,
           Wl_dp, Wr_dp, b_dp, Wl_pd, Wr_pd, b_pd, W_lin, b_lin):
    t8 = _dense8(x_drug, x_prot, Wl_dp, Wl_pd, Wr_dp, Wr_pd, W_lin,
                 b_dp, b_pd, b_lin)

    acc_dp, cnt_dp, acc_pd, cnt_pd = _segsum(
        edge_index_dp.astype(i32), edge_index_pd.astype(i32), t8)
    s_prot, s_drug = _finalize(acc_dp, cnt_dp, acc_pd, cnt_pd, t8)
    out = _edge_gather(s_drug, s_prot, edge_label_index.astype(i32))
    return out[:, None]
